# baseline (speedup 1.0000x reference)
import functools
import jax
import jax.numpy as jnp
from jax import lax
from jax.experimental import pallas as pl
from jax.experimental.pallas import tpu as pltpu
from jax.experimental.pallas import tpu_sc as plsc

N = 10000
NPAD = 10240
E = 160000
D = 128
NG = 64
TN = 2000
TE = 2000
CH = 40
EPT = E // 16
NCH = EPT // CH
RPT = NPAD // 16

F32 = jnp.float32


def _swish(x):
    return x * jax.nn.sigmoid(x)


def _dott(a, b, hi=False):
    return lax.dot_general(a, b, (((1,), (1,)), ((), ())),
                           preferred_element_type=F32,
                           precision=lax.Precision.HIGHEST if hi else None)


def _dot(a, b, hi=False):
    return lax.dot_general(a, b, (((1,), (0,)), ((), ())),
                           preferred_element_type=F32,
                           precision=lax.Precision.HIGHEST if hi else None)


def _dotT(a, b, hi=False):
    return lax.dot_general(a, b, (((0,), (0,)), ((), ())),
                           preferred_element_type=F32,
                           precision=lax.Precision.HIGHEST if hi else None)


def _onehot(idx, k):
    return (idx[:, None] == lax.broadcasted_iota(jnp.int32, (1, k), 1)).astype(F32)


def _embed_body(z_ref, emb_ref, x_ref):
    zt = z_ref[0, 0]
    oh = _onehot(zt, 95)
    x_ref[...] = _swish(_dot(oh, emb_ref[...], hi=True))


def _embed(z, emb):
    return pl.pallas_call(
        _embed_body,
        grid=(N // TN,),
        in_specs=[pl.BlockSpec((1, 1, TN), lambda j: (j, 0, 0)),
                  pl.BlockSpec((95, D), lambda j: (0, 0))],
        out_specs=pl.BlockSpec((TN, D), lambda j: (j, 0)),
        out_shape=jax.ShapeDtypeStruct((N, D), F32),
    )(z, emb)


def _hproj_body(x_ref, Wl_ref, bl_ref, h_ref):
    h_ref[...] = _swish(_dott(x_ref[...], Wl_ref[...]) + bl_ref[...])


def _hproj(x, Wl, bl):
    return pl.pallas_call(
        _hproj_body,
        grid=(N // TN,),
        in_specs=[pl.BlockSpec((TN, D), lambda j: (j, 0)),
                  pl.BlockSpec((D, D), lambda j: (0, 0)),
                  pl.BlockSpec((1, D), lambda j: (0, 0))],
        out_specs=pl.BlockSpec((TN, D), lambda j: (j, 0)),
        out_shape=jax.ShapeDtypeStruct((N, D), F32),
    )(x, Wl, bl)


def _fproj_body(f1_ref, f2_ref, Wf1a_ref, Wf1b_ref, Wf2a_ref, Wf2b_ref, o_ref):
    o_ref[0] = _dott(_dott(f1_ref[...], Wf1a_ref[...]), Wf1b_ref[...])
    o_ref[1] = _dott(_dott(f2_ref[...], Wf2a_ref[...]), Wf2b_ref[...])


def _fproj(feature1, feature2, Wf1a, Wf1b, Wf2a, Wf2b):
    return pl.pallas_call(
        _fproj_body,
        grid=(E // TE,),
        in_specs=[pl.BlockSpec((TE, 12), lambda j: (j, 0)),
                  pl.BlockSpec((TE, 6), lambda j: (j, 0)),
                  pl.BlockSpec((D, 12), lambda j: (0, 0)),
                  pl.BlockSpec((D, D), lambda j: (0, 0)),
                  pl.BlockSpec((D, 6), lambda j: (0, 0)),
                  pl.BlockSpec((D, D), lambda j: (0, 0))],
        out_specs=pl.BlockSpec((2, TE, D), lambda j: (0, j, 0)),
        out_shape=jax.ShapeDtypeStruct((2, E, D), F32),
    )(feature1, feature2, Wf1a, Wf1b, Wf2a, Wf2b)


def _agg_body(h_hbm, f_hbm, src_hbm, dst_hbm, out_hbm,
              srcall, dstb0, dstb1, dstb2, rows0, rows1, rows2,
              fbuf0, fbuf1, fbuf2, acc,
              sg0, sg1, sg2, sf0, sf1, sf2, sd0, sd1, sd2):
    cid = lax.axis_index("c")
    sid = lax.axis_index("s")
    tbase = sid * EPT
    dstb = (dstb0, dstb1, dstb2)
    rows = (rows0, rows1, rows2)
    fbuf = (fbuf0, fbuf1, fbuf2)
    sg = (sg0, sg1, sg2)
    sf = (sf0, sf1, sf2)
    sd = (sd0, sd1, sd2)

    zero16 = jnp.zeros((16,), F32)

    def zrow(r, _):
        for c in range(8):
            fbuf0[r, pl.ds(c * 16, 16)] = zero16
        return 0
    lax.fori_loop(0, CH, zrow, 0)
    for k in range(RPT // CH):
        pltpu.sync_copy(fbuf0, acc.at[pl.ds(sid * RPT + k * CH, CH)])

    pltpu.sync_copy(src_hbm.at[pl.ds(tbase, EPT)], srcall)
    plsc.subcore_barrier()

    def prefetch(p, i):
        base = tbase + i * CH
        pltpu.async_copy(dst_hbm.at[pl.ds(base, CH)], dstb[p], sd[p])
        pltpu.async_copy(f_hbm.at[cid, pl.ds(base, CH)], fbuf[p], sf[p])
        pltpu.async_copy(h_hbm.at[srcall.at[pl.ds(i * CH, CH)]],
                         rows[p], sg[p])

    def process(p, i):
        base = tbase + i * CH
        pltpu.make_async_copy(dst_hbm.at[pl.ds(base, CH)], dstb[p],
                              sd[p]).wait()
        pltpu.make_async_copy(f_hbm.at[cid, pl.ds(base, CH)], fbuf[p],
                              sf[p]).wait()
        pltpu.make_async_copy(h_hbm.at[srcall.at[pl.ds(i * CH, CH)]],
                              rows[p], sg[p]).wait()

        def mrow(r, _):
            for c in range(8):
                s = pl.ds(c * 16, 16)
                rows[p][r, s] = rows[p][r, s] * fbuf[p][r, s]
            return 0
        lax.fori_loop(0, CH, mrow, 0, unroll=5)
        pltpu.sync_copy(rows[p], acc.at[dstb[p]], add=True)

    prefetch(0, 0)
    prefetch(1, 1)

    def chunk(i, _):
        for p in range(3):
            @pl.when(i % 3 == p)
            def _():
                @pl.when(i + 2 < NCH)
                def _():
                    prefetch((p + 2) % 3, i + 2)
                process(p, i)
        return 0
    lax.fori_loop(0, NCH, chunk, 0)

    plsc.subcore_barrier()
    pltpu.sync_copy(acc.at[pl.ds(sid * RPT, RPT)],
                    out_hbm.at[cid, pl.ds(sid * RPT, RPT)])


@functools.cache
def _agg_call():
    return pl.kernel(
        _agg_body,
        out_type=jax.ShapeDtypeStruct((2, NPAD, D), F32),
        mesh=plsc.VectorSubcoreMesh(core_axis_name="c", subcore_axis_name="s"),
        scratch_types=[
            pltpu.VMEM((EPT,), jnp.int32),
            pltpu.VMEM((CH,), jnp.int32),
            pltpu.VMEM((CH,), jnp.int32),
            pltpu.VMEM((CH,), jnp.int32),
            pltpu.VMEM((CH, D), F32),
            pltpu.VMEM((CH, D), F32),
            pltpu.VMEM((CH, D), F32),
            pltpu.VMEM((CH, D), F32),
            pltpu.VMEM((CH, D), F32),
            pltpu.VMEM((CH, D), F32),
            pltpu.VMEM_SHARED((NPAD, D), F32),
            pltpu.SemaphoreType.DMA,
            pltpu.SemaphoreType.DMA,
            pltpu.SemaphoreType.DMA,
            pltpu.SemaphoreType.DMA,
            pltpu.SemaphoreType.DMA,
            pltpu.SemaphoreType.DMA,
            pltpu.SemaphoreType.DMA,
            pltpu.SemaphoreType.DMA,
            pltpu.SemaphoreType.DMA,
        ],
    )


def _edge_agg(h, fstk, src, dst):
    return _agg_call()(h, fstk, src, dst)


def _block_body(agg_ref, h_ref, batch_ref,
                Wc1l_ref, bc1l_ref, Wc1r_ref, W1_ref, b1_ref,
                Wc2l_ref, bc2l_ref, Wc2r_ref, W2_ref, b2_ref,
                Wcat_ref, bcat_ref, Wres_ref, bres_ref,
                hh_ref, s1_ref, c_ref):
    j = pl.program_id(0)
    h = h_ref[...]
    h1 = _dott(agg_ref[0], Wc1l_ref[...]) + bc1l_ref[...] + _dott(h, Wc1r_ref[...])
    h1 = _swish(_dott(h1, W1_ref[...]) + b1_ref[...])
    h2 = _dott(agg_ref[1], Wc2l_ref[...]) + bc2l_ref[...] + _dott(h, Wc2r_ref[...])
    h2 = _swish(_dott(h2, W2_ref[...]) + b2_ref[...])
    hh = (_dott(jnp.concatenate([h1, h2], 1), Wcat_ref[...])
          + bcat_ref[...] + h)
    for r in range(3):
        hh = _swish(_dott(hh, Wres_ref[r]) + bres_ref[r][None, :]) + hh
    hh_ref[...] = hh

    bt = batch_ref[0, 0]
    oh = _onehot(bt, NG)
    p1 = _dotT(oh, hh, hi=True)
    pc = _dotT(oh, jnp.ones_like(hh), hi=True)

    @pl.when(j == 0)
    def _():
        s1_ref[...] = p1
        c_ref[...] = pc

    @pl.when(j > 0)
    def _():
        s1_ref[...] += p1
        c_ref[...] += pc


def _block_post(agg, h, batch, Wc1l, bc1l, Wc1r, W1, b1,
                Wc2l, bc2l, Wc2r, W2, b2, Wcat, bcat, Wres, bres):
    full = lambda shape: pl.BlockSpec(shape, lambda j: tuple(0 for _ in shape))
    return pl.pallas_call(
        _block_body,
        grid=(N // TN,),
        in_specs=[pl.BlockSpec((2, TN, D), lambda j: (0, j, 0)),
                  pl.BlockSpec((TN, D), lambda j: (j, 0)),
                  pl.BlockSpec((1, 1, TN), lambda j: (j, 0, 0)),
                  full((D, D)), full((1, D)), full((D, D)),
                  full((D, D)), full((1, D)),
                  full((D, D)), full((1, D)), full((D, D)),
                  full((D, D)), full((1, D)),
                  full((D, 2 * D)), full((1, D)),
                  full((3, D, D)), full((3, D))],
        out_specs=[pl.BlockSpec((TN, D), lambda j: (j, 0)),
                   pl.BlockSpec((NG, D), lambda j: (0, 0)),
                   pl.BlockSpec((NG, D), lambda j: (0, 0))],
        out_shape=[jax.ShapeDtypeStruct((N, D), F32),
                   jax.ShapeDtypeStruct((NG, D), F32),
                   jax.ShapeDtypeStruct((NG, D), F32)],
    )(agg, h, batch, Wc1l, bc1l, Wc1r, W1, b1,
      Wc2l, bc2l, Wc2r, W2, b2, Wcat, bcat, Wres, bres)


def _varsum_body(hh_ref, s1_ref, c_ref, batch_ref, alpha_ref, s2_ref):
    j = pl.program_id(0)
    cnt = jnp.maximum(c_ref[...], 1.0)
    mean = s1_ref[...] / cnt
    bt = batch_ref[0, 0]
    oh = _onehot(bt, NG)
    sub = hh_ref[...] - alpha_ref[...] * _dot(oh, mean, hi=True)
    p2 = _dotT(oh, sub * sub, hi=True)

    @pl.when(j == 0)
    def _():
        s2_ref[...] = p2

    @pl.when(j > 0)
    def _():
        s2_ref[...] += p2


def _varsum(hh, s1, cntf, batch, alpha):
    full = lambda shape: pl.BlockSpec(shape, lambda j: tuple(0 for _ in shape))
    return pl.pallas_call(
        _varsum_body,
        grid=(N // TN,),
        in_specs=[pl.BlockSpec((TN, D), lambda j: (j, 0)),
                  full((NG, D)), full((NG, D)),
                  pl.BlockSpec((1, 1, TN), lambda j: (j, 0, 0)),
                  full((1, D))],
        out_specs=pl.BlockSpec((NG, D), lambda j: (0, 0)),
        out_shape=jax.ShapeDtypeStruct((NG, D), F32),
    )(hh, s1, cntf, batch, alpha)


def _norm_body(hh_ref, s1_ref, s2_ref, c_ref, batch_ref,
               alpha_ref, gamma_ref, beta_ref, Wfin_ref, bfin_ref, x_ref):
    cnt = jnp.maximum(c_ref[...], 1.0)
    mean = s1_ref[...] / cnt
    var = s2_ref[...] / cnt
    al = alpha_ref[...]
    bt = batch_ref[0, 0]
    oh = _onehot(bt, NG)
    mrow = _dot(oh, mean, hi=True)
    vrow = _dot(oh, var, hi=True)
    hh = hh_ref[...]
    nrm = (gamma_ref[...] * (hh - al * mrow) / jnp.sqrt(vrow + 1e-5)
           + beta_ref[...])
    x_ref[...] = _dott(nrm, Wfin_ref[...]) + bfin_ref[...]


def _norm_fin(hh, s1, s2, cntf, batch, alpha, gamma, beta, Wfin, bfin):
    full = lambda shape: pl.BlockSpec(shape, lambda j: tuple(0 for _ in shape))
    return pl.pallas_call(
        _norm_body,
        grid=(N // TN,),
        in_specs=[pl.BlockSpec((TN, D), lambda j: (j, 0)),
                  full((NG, D)), full((NG, D)), full((NG, D)),
                  pl.BlockSpec((1, 1, TN), lambda j: (j, 0, 0)),
                  full((1, D)), full((1, D)), full((1, D)),
                  full((D, D)), full((1, D))],
        out_specs=pl.BlockSpec((TN, D), lambda j: (j, 0)),
        out_shape=jax.ShapeDtypeStruct((N, D), F32),
    )(hh, s1, s2, cntf, batch, alpha, gamma, beta, Wfin, bfin)


def _final_body(x_ref, batch_ref, Wout_ref, bout_ref, Wlast_ref, blast_ref,
                c_ref, energy_ref, sacc):
    j = pl.program_id(0)
    t = x_ref[...]
    for r in range(3):
        t = _swish(_dott(t, Wout_ref[r]) + bout_ref[r][None, :])
    bt = batch_ref[0, 0]
    oh = _onehot(bt, NG)
    e = _dott(t, Wlast_ref[...])
    p = _dotT(oh, e, hi=True)

    @pl.when(j == 0)
    def _():
        sacc[...] = p

    @pl.when(j > 0)
    def _():
        sacc[...] += p

    @pl.when(j == pl.num_programs(0) - 1)
    def _():
        cnt = jnp.maximum(c_ref[...], 1.0)
        energy_ref[...] = sacc[...] + _dott(cnt, blast_ref[...], hi=True)


def _final(x, batch, Wout, bout, Wlast, blast, cntf):
    full = lambda shape: pl.BlockSpec(shape, lambda j: tuple(0 for _ in shape))
    return pl.pallas_call(
        _final_body,
        grid=(N // TN,),
        in_specs=[pl.BlockSpec((TN, D), lambda j: (j, 0)),
                  pl.BlockSpec((1, 1, TN), lambda j: (j, 0, 0)),
                  full((3, D, D)), full((3, D)),
                  full((1, D)), full((1, D)), full((NG, D))],
        out_specs=pl.BlockSpec((NG, 1), lambda j: (0, 0)),
        out_shape=jax.ShapeDtypeStruct((NG, 1), F32),
        scratch_shapes=[pltpu.VMEM((NG, 1), F32)],
    )(x, batch, Wout, bout, Wlast, blast, cntf)


def kernel(emb, feature1, feature2, Wl, bl, Wf1a, Wf1b, Wf2a, Wf2b,
           Wc1l, bc1l, Wc1r, Wc2l, bc2l, Wc2r, W1, b1, W2, b2, Wcat, bcat,
           gamma, beta, alpha, Wres, bres, Wfin, bfin, Wout, bout,
           Wlast, blast, z, edge_index, batch):
    z = z.astype(jnp.int32).reshape(N // TN, 1, TN)
    src = edge_index[0].astype(jnp.int32)
    dst = edge_index[1].astype(jnp.int32)
    batch = batch.astype(jnp.int32).reshape(N // TN, 1, TN)
    r1 = lambda v: v.reshape(1, D)

    x = _embed(z, emb)
    cntf = None
    for i in range(2):
        h = _hproj(x, Wl[i], r1(bl[i]))
        fstk = _fproj(feature1, feature2, Wf1a[i], Wf1b[i], Wf2a[i], Wf2b[i])
        agg = _edge_agg(h, fstk, src, dst)
        hh, s1, cntf = _block_post(
            agg, h, batch, Wc1l[i], r1(bc1l[i]), Wc1r[i], W1[i], r1(b1[i]),
            Wc2l[i], r1(bc2l[i]), Wc2r[i], W2[i], r1(b2[i]),
            Wcat[i], r1(bcat[i]), Wres[i], bres[i])
        s2 = _varsum(hh, s1, cntf, batch, r1(alpha[i]))
        x = _norm_fin(hh, s1, s2, cntf, batch, r1(alpha[i]), r1(gamma[i]),
                      r1(beta[i]), Wfin[i], r1(bfin[i]))
    blastv = jnp.broadcast_to(blast.reshape(1, 1) / D, (1, D))
    return _final(x, batch, Wout, bout, Wlast, blastv, cntf)


# speedup vs baseline: 1.4584x; 1.4584x over previous
import functools
import jax
import jax.numpy as jnp
from jax import lax
from jax.experimental import pallas as pl
from jax.experimental.pallas import tpu as pltpu
from jax.experimental.pallas import tpu_sc as plsc

N = 10000
NPAD = 10240
E = 160000
D = 128
NG = 64
TN = 2000
TE = 2000
CH = 40
EPT = E // 16
NCH = EPT // CH
RPT = NPAD // 16

F32 = jnp.float32


def _swish(x):
    return x * jax.nn.sigmoid(x)


def _dott(a, b, hi=False):
    return lax.dot_general(a, b, (((1,), (1,)), ((), ())),
                           preferred_element_type=F32,
                           precision=lax.Precision.HIGHEST if hi else None)


def _dot(a, b, hi=False):
    return lax.dot_general(a, b, (((1,), (0,)), ((), ())),
                           preferred_element_type=F32,
                           precision=lax.Precision.HIGHEST if hi else None)


def _dotT(a, b, hi=False):
    return lax.dot_general(a, b, (((0,), (0,)), ((), ())),
                           preferred_element_type=F32,
                           precision=lax.Precision.HIGHEST if hi else None)


def _onehot(idx, k):
    return (idx[:, None] == lax.broadcasted_iota(jnp.int32, (1, k), 1)).astype(F32)


def _embed_body(z_ref, emb_ref, x_ref):
    zt = z_ref[0, 0]
    oh = _onehot(zt, 95)
    x_ref[...] = _swish(_dot(oh, emb_ref[...], hi=True))


def _embed(z, emb):
    return pl.pallas_call(
        _embed_body,
        grid=(N // TN,),
        in_specs=[pl.BlockSpec((1, 1, TN), lambda j: (j, 0, 0)),
                  pl.BlockSpec((95, D), lambda j: (0, 0))],
        out_specs=pl.BlockSpec((TN, D), lambda j: (j, 0)),
        out_shape=jax.ShapeDtypeStruct((N, D), F32),
    )(z, emb)


def _hproj_body(x_ref, Wl_ref, bl_ref, h_ref):
    h_ref[...] = _swish(_dott(x_ref[...], Wl_ref[...]) + bl_ref[...])


def _hproj(x, Wl, bl):
    return pl.pallas_call(
        _hproj_body,
        grid=(N // TN,),
        in_specs=[pl.BlockSpec((TN, D), lambda j: (j, 0)),
                  pl.BlockSpec((D, D), lambda j: (0, 0)),
                  pl.BlockSpec((1, D), lambda j: (0, 0))],
        out_specs=pl.BlockSpec((TN, D), lambda j: (j, 0)),
        out_shape=jax.ShapeDtypeStruct((N, D), F32),
    )(x, Wl, bl)


def _fproj_body(f1_ref, f2_ref, Wf1a_ref, Wf1b_ref, Wf2a_ref, Wf2b_ref, o_ref):
    o_ref[0] = _dott(_dott(f1_ref[...], Wf1a_ref[...]), Wf1b_ref[...])
    o_ref[1] = _dott(_dott(f2_ref[...], Wf2a_ref[...]), Wf2b_ref[...])


def _fproj(feature1, feature2, Wf1a, Wf1b, Wf2a, Wf2b):
    return pl.pallas_call(
        _fproj_body,
        grid=(E // TE,),
        in_specs=[pl.BlockSpec((TE, 12), lambda j: (j, 0)),
                  pl.BlockSpec((TE, 6), lambda j: (j, 0)),
                  pl.BlockSpec((D, 12), lambda j: (0, 0)),
                  pl.BlockSpec((D, D), lambda j: (0, 0)),
                  pl.BlockSpec((D, 6), lambda j: (0, 0)),
                  pl.BlockSpec((D, D), lambda j: (0, 0))],
        out_specs=pl.BlockSpec((2, TE, D), lambda j: (0, j, 0)),
        out_shape=jax.ShapeDtypeStruct((2, E, D), F32),
    )(feature1, feature2, Wf1a, Wf1b, Wf2a, Wf2b)


def _agg_body(h_hbm, f_hbm, src_hbm, dst_hbm, out_hbm,
              srcall, dstb0, dstb1, rows0, rows1, fbuf0, fbuf1, acc,
              sg0, sg1, sf0, sf1, sd0, sd1):
    cid = lax.axis_index("c")
    sid = lax.axis_index("s")
    tbase = sid * EPT
    dstb = (dstb0, dstb1)
    rows = (rows0, rows1)
    fbuf = (fbuf0, fbuf1)
    sg = (sg0, sg1)
    sf = (sf0, sf1)
    sd = (sd0, sd1)

    zero16 = jnp.zeros((16,), F32)

    def zrow(r, _):
        for c in range(8):
            fbuf0[r, pl.ds(c * 16, 16)] = zero16
        return 0
    lax.fori_loop(0, CH, zrow, 0)
    for k in range(RPT // CH):
        pltpu.sync_copy(fbuf0, acc.at[pl.ds(sid * RPT + k * CH, CH)])

    pltpu.sync_copy(src_hbm.at[pl.ds(tbase, EPT)], srcall)
    plsc.subcore_barrier()

    def prefetch(p, i):
        base = tbase + i * CH
        pltpu.async_copy(dst_hbm.at[pl.ds(base, CH)], dstb[p], sd[p])
        pltpu.async_copy(f_hbm.at[cid, pl.ds(base, CH)], fbuf[p], sf[p])
        pltpu.async_copy(h_hbm.at[srcall.at[pl.ds(i * CH, CH)]],
                         rows[p], sg[p])

    def process(p, i):
        base = tbase + i * CH
        pltpu.make_async_copy(dst_hbm.at[pl.ds(base, CH)], dstb[p],
                              sd[p]).wait()
        pltpu.make_async_copy(f_hbm.at[cid, pl.ds(base, CH)], fbuf[p],
                              sf[p]).wait()
        pltpu.make_async_copy(h_hbm.at[srcall.at[pl.ds(i * CH, CH)]],
                              rows[p], sg[p]).wait()

        def mrow(r, _):
            for c in range(8):
                s = pl.ds(c * 16, 16)
                rows[p][r, s] = rows[p][r, s] * fbuf[p][r, s]
            return 0
        lax.fori_loop(0, CH, mrow, 0)
        pltpu.sync_copy(rows[p], acc.at[dstb[p]], add=True)

    prefetch(0, 0)

    def chunk(i, _):
        for p in range(2):
            @pl.when(i % 2 == p)
            def _():
                @pl.when(i + 1 < NCH)
                def _():
                    prefetch(1 - p, i + 1)
                process(p, i)
        return 0
    lax.fori_loop(0, NCH, chunk, 0)

    plsc.subcore_barrier()
    pltpu.sync_copy(acc.at[pl.ds(sid * RPT, RPT)],
                    out_hbm.at[cid, pl.ds(sid * RPT, RPT)])


@functools.cache
def _agg_call():
    return pl.kernel(
        _agg_body,
        out_type=jax.ShapeDtypeStruct((2, NPAD, D), F32),
        mesh=plsc.VectorSubcoreMesh(core_axis_name="c", subcore_axis_name="s"),
        scratch_types=[
            pltpu.VMEM((EPT,), jnp.int32),
            pltpu.VMEM((CH,), jnp.int32),
            pltpu.VMEM((CH,), jnp.int32),
            pltpu.VMEM((CH, D), F32),
            pltpu.VMEM((CH, D), F32),
            pltpu.VMEM((CH, D), F32),
            pltpu.VMEM((CH, D), F32),
            pltpu.VMEM_SHARED((NPAD, D), F32),
            pltpu.SemaphoreType.DMA,
            pltpu.SemaphoreType.DMA,
            pltpu.SemaphoreType.DMA,
            pltpu.SemaphoreType.DMA,
            pltpu.SemaphoreType.DMA,
            pltpu.SemaphoreType.DMA,
        ],
    )


def _edge_agg(h, fstk, src, dst):
    return _agg_call()(h, fstk, src, dst)


def _block_body(agg_ref, h_ref, batch_ref,
                Wc1l_ref, bc1l_ref, Wc1r_ref, W1_ref, b1_ref,
                Wc2l_ref, bc2l_ref, Wc2r_ref, W2_ref, b2_ref,
                Wcat_ref, bcat_ref, Wres_ref, bres_ref,
                hh_ref, s1_ref, c_ref):
    j = pl.program_id(0)
    h = h_ref[...]
    h1 = _dott(agg_ref[0], Wc1l_ref[...]) + bc1l_ref[...] + _dott(h, Wc1r_ref[...])
    h1 = _swish(_dott(h1, W1_ref[...]) + b1_ref[...])
    h2 = _dott(agg_ref[1], Wc2l_ref[...]) + bc2l_ref[...] + _dott(h, Wc2r_ref[...])
    h2 = _swish(_dott(h2, W2_ref[...]) + b2_ref[...])
    hh = (_dott(jnp.concatenate([h1, h2], 1), Wcat_ref[...])
          + bcat_ref[...] + h)
    for r in range(3):
        hh = _swish(_dott(hh, Wres_ref[r]) + bres_ref[r][None, :]) + hh
    hh_ref[...] = hh

    bt = batch_ref[0, 0]
    oh = _onehot(bt, NG)
    p1 = _dotT(oh, hh, hi=True)
    pc = _dotT(oh, jnp.ones_like(hh), hi=True)

    @pl.when(j == 0)
    def _():
        s1_ref[...] = p1
        c_ref[...] = pc

    @pl.when(j > 0)
    def _():
        s1_ref[...] += p1
        c_ref[...] += pc


def _block_post(agg, h, batch, Wc1l, bc1l, Wc1r, W1, b1,
                Wc2l, bc2l, Wc2r, W2, b2, Wcat, bcat, Wres, bres):
    full = lambda shape: pl.BlockSpec(shape, lambda j: tuple(0 for _ in shape))
    return pl.pallas_call(
        _block_body,
        grid=(N // TN,),
        in_specs=[pl.BlockSpec((2, TN, D), lambda j: (0, j, 0)),
                  pl.BlockSpec((TN, D), lambda j: (j, 0)),
                  pl.BlockSpec((1, 1, TN), lambda j: (j, 0, 0)),
                  full((D, D)), full((1, D)), full((D, D)),
                  full((D, D)), full((1, D)),
                  full((D, D)), full((1, D)), full((D, D)),
                  full((D, D)), full((1, D)),
                  full((D, 2 * D)), full((1, D)),
                  full((3, D, D)), full((3, D))],
        out_specs=[pl.BlockSpec((TN, D), lambda j: (j, 0)),
                   pl.BlockSpec((NG, D), lambda j: (0, 0)),
                   pl.BlockSpec((NG, D), lambda j: (0, 0))],
        out_shape=[jax.ShapeDtypeStruct((N, D), F32),
                   jax.ShapeDtypeStruct((NG, D), F32),
                   jax.ShapeDtypeStruct((NG, D), F32)],
    )(agg, h, batch, Wc1l, bc1l, Wc1r, W1, b1,
      Wc2l, bc2l, Wc2r, W2, b2, Wcat, bcat, Wres, bres)


def _varsum_body(hh_ref, s1_ref, c_ref, batch_ref, alpha_ref, s2_ref):
    j = pl.program_id(0)
    cnt = jnp.maximum(c_ref[...], 1.0)
    mean = s1_ref[...] / cnt
    bt = batch_ref[0, 0]
    oh = _onehot(bt, NG)
    sub = hh_ref[...] - alpha_ref[...] * _dot(oh, mean, hi=True)
    p2 = _dotT(oh, sub * sub, hi=True)

    @pl.when(j == 0)
    def _():
        s2_ref[...] = p2

    @pl.when(j > 0)
    def _():
        s2_ref[...] += p2


def _varsum(hh, s1, cntf, batch, alpha):
    full = lambda shape: pl.BlockSpec(shape, lambda j: tuple(0 for _ in shape))
    return pl.pallas_call(
        _varsum_body,
        grid=(N // TN,),
        in_specs=[pl.BlockSpec((TN, D), lambda j: (j, 0)),
                  full((NG, D)), full((NG, D)),
                  pl.BlockSpec((1, 1, TN), lambda j: (j, 0, 0)),
                  full((1, D))],
        out_specs=pl.BlockSpec((NG, D), lambda j: (0, 0)),
        out_shape=jax.ShapeDtypeStruct((NG, D), F32),
    )(hh, s1, cntf, batch, alpha)


def _norm_body(hh_ref, s1_ref, s2_ref, c_ref, batch_ref,
               alpha_ref, gamma_ref, beta_ref, Wfin_ref, bfin_ref, x_ref):
    cnt = jnp.maximum(c_ref[...], 1.0)
    mean = s1_ref[...] / cnt
    var = s2_ref[...] / cnt
    al = alpha_ref[...]
    bt = batch_ref[0, 0]
    oh = _onehot(bt, NG)
    mrow = _dot(oh, mean, hi=True)
    vrow = _dot(oh, var, hi=True)
    hh = hh_ref[...]
    nrm = (gamma_ref[...] * (hh - al * mrow) / jnp.sqrt(vrow + 1e-5)
           + beta_ref[...])
    x_ref[...] = _dott(nrm, Wfin_ref[...]) + bfin_ref[...]


def _norm_fin(hh, s1, s2, cntf, batch, alpha, gamma, beta, Wfin, bfin):
    full = lambda shape: pl.BlockSpec(shape, lambda j: tuple(0 for _ in shape))
    return pl.pallas_call(
        _norm_body,
        grid=(N // TN,),
        in_specs=[pl.BlockSpec((TN, D), lambda j: (j, 0)),
                  full((NG, D)), full((NG, D)), full((NG, D)),
                  pl.BlockSpec((1, 1, TN), lambda j: (j, 0, 0)),
                  full((1, D)), full((1, D)), full((1, D)),
                  full((D, D)), full((1, D))],
        out_specs=pl.BlockSpec((TN, D), lambda j: (j, 0)),
        out_shape=jax.ShapeDtypeStruct((N, D), F32),
    )(hh, s1, s2, cntf, batch, alpha, gamma, beta, Wfin, bfin)


def _final_body(x_ref, batch_ref, Wout_ref, bout_ref, Wlast_ref, blast_ref,
                c_ref, energy_ref, sacc):
    j = pl.program_id(0)
    t = x_ref[...]
    for r in range(3):
        t = _swish(_dott(t, Wout_ref[r]) + bout_ref[r][None, :])
    bt = batch_ref[0, 0]
    oh = _onehot(bt, NG)
    e = _dott(t, Wlast_ref[...])
    p = _dotT(oh, e, hi=True)

    @pl.when(j == 0)
    def _():
        sacc[...] = p

    @pl.when(j > 0)
    def _():
        sacc[...] += p

    @pl.when(j == pl.num_programs(0) - 1)
    def _():
        cnt = jnp.maximum(c_ref[...], 1.0)
        energy_ref[...] = sacc[...] + _dott(cnt, blast_ref[...], hi=True)


def _final(x, batch, Wout, bout, Wlast, blast, cntf):
    full = lambda shape: pl.BlockSpec(shape, lambda j: tuple(0 for _ in shape))
    return pl.pallas_call(
        _final_body,
        grid=(N // TN,),
        in_specs=[pl.BlockSpec((TN, D), lambda j: (j, 0)),
                  pl.BlockSpec((1, 1, TN), lambda j: (j, 0, 0)),
                  full((3, D, D)), full((3, D)),
                  full((1, D)), full((1, D)), full((NG, D))],
        out_specs=pl.BlockSpec((NG, 1), lambda j: (0, 0)),
        out_shape=jax.ShapeDtypeStruct((NG, 1), F32),
        scratch_shapes=[pltpu.VMEM((NG, 1), F32)],
    )(x, batch, Wout, bout, Wlast, blast, cntf)


def kernel(emb, feature1, feature2, Wl, bl, Wf1a, Wf1b, Wf2a, Wf2b,
           Wc1l, bc1l, Wc1r, Wc2l, bc2l, Wc2r, W1, b1, W2, b2, Wcat, bcat,
           gamma, beta, alpha, Wres, bres, Wfin, bfin, Wout, bout,
           Wlast, blast, z, edge_index, batch):
    z = z.astype(jnp.int32).reshape(N // TN, 1, TN)
    src = edge_index[0].astype(jnp.int32)
    dst = edge_index[1].astype(jnp.int32)
    batch = batch.astype(jnp.int32).reshape(N // TN, 1, TN)
    r1 = lambda v: v.reshape(1, D)

    x = _embed(z, emb)
    cntf = None
    for i in range(2):
        h = _hproj(x, Wl[i], r1(bl[i]))
        fstk = _fproj(feature1, feature2, Wf1a[i], Wf1b[i], Wf2a[i], Wf2b[i])
        agg = _edge_agg(h, fstk, src, dst)
        hh, s1, cntf = _block_post(
            agg, h, batch, Wc1l[i], r1(bc1l[i]), Wc1r[i], W1[i], r1(b1[i]),
            Wc2l[i], r1(bc2l[i]), Wc2r[i], W2[i], r1(b2[i]),
            Wcat[i], r1(bcat[i]), Wres[i], bres[i])
        s2 = _varsum(hh, s1, cntf, batch, r1(alpha[i]))
        x = _norm_fin(hh, s1, s2, cntf, batch, r1(alpha[i]), r1(gamma[i]),
                      r1(beta[i]), Wfin[i], r1(bfin[i]))
    blastv = jnp.broadcast_to(blast.reshape(1, 1) / D, (1, D))
    return _final(x, batch, Wout, bout, Wlast, blastv, cntf)


# speedup vs baseline: 1.4732x; 1.0102x over previous
import functools
import jax
import jax.numpy as jnp
from jax import lax
from jax.experimental import pallas as pl
from jax.experimental.pallas import tpu as pltpu
from jax.experimental.pallas import tpu_sc as plsc

N = 10000
NPAD = 10240
E = 160000
D = 128
NG = 64
TN = 2000
TE = 2000
CH = 40
EPT = E // 16
NCH = EPT // CH
RPT = NPAD // 16

F32 = jnp.float32


def _swish(x):
    return x * jax.nn.sigmoid(x)


def _dott(a, b, hi=False):
    return lax.dot_general(a, b, (((1,), (1,)), ((), ())),
                           preferred_element_type=F32,
                           precision=lax.Precision.HIGHEST if hi else None)


def _dot(a, b, hi=False):
    return lax.dot_general(a, b, (((1,), (0,)), ((), ())),
                           preferred_element_type=F32,
                           precision=lax.Precision.HIGHEST if hi else None)


def _dotT(a, b, hi=False):
    return lax.dot_general(a, b, (((0,), (0,)), ((), ())),
                           preferred_element_type=F32,
                           precision=lax.Precision.HIGHEST if hi else None)


def _onehot(idx, k):
    return (idx[:, None] == lax.broadcasted_iota(jnp.int32, (1, k), 1)).astype(F32)


def _embed_body(z_ref, emb_ref, Wl_ref, bl_ref, x_ref, h_ref):
    zt = z_ref[0, 0]
    oh = _onehot(zt, 95)
    x = _swish(_dot(oh, emb_ref[...], hi=True))
    x_ref[...] = x
    h_ref[...] = _swish(_dott(x, Wl_ref[...]) + bl_ref[...])


def _embed(z, emb, Wl, bl):
    return pl.pallas_call(
        _embed_body,
        grid=(N // TN,),
        in_specs=[pl.BlockSpec((1, 1, TN), lambda j: (j, 0, 0)),
                  pl.BlockSpec((95, D), lambda j: (0, 0)),
                  pl.BlockSpec((D, D), lambda j: (0, 0)),
                  pl.BlockSpec((1, D), lambda j: (0, 0))],
        out_specs=[pl.BlockSpec((TN, D), lambda j: (j, 0)),
                   pl.BlockSpec((TN, D), lambda j: (j, 0))],
        out_shape=[jax.ShapeDtypeStruct((N, D), F32),
                   jax.ShapeDtypeStruct((N, D), F32)],
    )(z, emb, Wl, bl)


def _hproj_body(x_ref, Wl_ref, bl_ref, h_ref):
    h_ref[...] = _swish(_dott(x_ref[...], Wl_ref[...]) + bl_ref[...])


def _hproj(x, Wl, bl):
    return pl.pallas_call(
        _hproj_body,
        grid=(N // TN,),
        in_specs=[pl.BlockSpec((TN, D), lambda j: (j, 0)),
                  pl.BlockSpec((D, D), lambda j: (0, 0)),
                  pl.BlockSpec((1, D), lambda j: (0, 0))],
        out_specs=pl.BlockSpec((TN, D), lambda j: (j, 0)),
        out_shape=jax.ShapeDtypeStruct((N, D), F32),
    )(x, Wl, bl)


def _fproj_body(f1_ref, f2_ref, Wf1a_ref, Wf1b_ref, Wf2a_ref, Wf2b_ref, o_ref):
    o_ref[0] = _dott(_dott(f1_ref[...], Wf1a_ref[...]), Wf1b_ref[...])
    o_ref[1] = _dott(_dott(f2_ref[...], Wf2a_ref[...]), Wf2b_ref[...])


def _fproj(feature1, feature2, Wf1a, Wf1b, Wf2a, Wf2b):
    return pl.pallas_call(
        _fproj_body,
        grid=(E // TE,),
        in_specs=[pl.BlockSpec((TE, 12), lambda j: (j, 0)),
                  pl.BlockSpec((TE, 6), lambda j: (j, 0)),
                  pl.BlockSpec((D, 12), lambda j: (0, 0)),
                  pl.BlockSpec((D, D), lambda j: (0, 0)),
                  pl.BlockSpec((D, 6), lambda j: (0, 0)),
                  pl.BlockSpec((D, D), lambda j: (0, 0))],
        out_specs=pl.BlockSpec((2, TE, D), lambda j: (0, j, 0)),
        out_shape=jax.ShapeDtypeStruct((2, E, D), F32),
    )(feature1, feature2, Wf1a, Wf1b, Wf2a, Wf2b)


def _agg_body(h_hbm, f_hbm, src_hbm, dst_hbm, out_hbm,
              srcall, dstb0, dstb1, rows0, rows1, fbuf0, fbuf1, acc,
              sg0, sg1, sf0, sf1, sd0, sd1):
    cid = lax.axis_index("c")
    sid = lax.axis_index("s")
    tbase = sid * EPT
    dstb = (dstb0, dstb1)
    rows = (rows0, rows1)
    fbuf = (fbuf0, fbuf1)
    sg = (sg0, sg1)
    sf = (sf0, sf1)
    sd = (sd0, sd1)

    zero16 = jnp.zeros((16,), F32)

    def zrow(r, _):
        for c in range(8):
            fbuf0[r, pl.ds(c * 16, 16)] = zero16
        return 0
    lax.fori_loop(0, CH, zrow, 0)
    for k in range(RPT // CH):
        pltpu.sync_copy(fbuf0, acc.at[pl.ds(sid * RPT + k * CH, CH)])

    pltpu.sync_copy(src_hbm.at[pl.ds(tbase, EPT)], srcall)
    plsc.subcore_barrier()

    def prefetch(p, i):
        base = tbase + i * CH
        pltpu.async_copy(dst_hbm.at[pl.ds(base, CH)], dstb[p], sd[p])
        pltpu.async_copy(f_hbm.at[cid, pl.ds(base, CH)], fbuf[p], sf[p])
        pltpu.async_copy(h_hbm.at[srcall.at[pl.ds(i * CH, CH)]],
                         rows[p], sg[p])

    def process(p, i):
        base = tbase + i * CH
        pltpu.make_async_copy(dst_hbm.at[pl.ds(base, CH)], dstb[p],
                              sd[p]).wait()
        pltpu.make_async_copy(f_hbm.at[cid, pl.ds(base, CH)], fbuf[p],
                              sf[p]).wait()
        pltpu.make_async_copy(h_hbm.at[srcall.at[pl.ds(i * CH, CH)]],
                              rows[p], sg[p]).wait()

        def mrow(r, _):
            for c in range(8):
                s = pl.ds(c * 16, 16)
                rows[p][r, s] = rows[p][r, s] * fbuf[p][r, s]
            return 0
        lax.fori_loop(0, CH, mrow, 0)
        pltpu.sync_copy(rows[p], acc.at[dstb[p]], add=True)

    prefetch(0, 0)

    def chunk(i, _):
        for p in range(2):
            @pl.when(i % 2 == p)
            def _():
                @pl.when(i + 1 < NCH)
                def _():
                    prefetch(1 - p, i + 1)
                process(p, i)
        return 0
    lax.fori_loop(0, NCH, chunk, 0)

    plsc.subcore_barrier()
    pltpu.sync_copy(acc.at[pl.ds(sid * RPT, RPT)],
                    out_hbm.at[cid, pl.ds(sid * RPT, RPT)])


@functools.cache
def _agg_call():
    return pl.kernel(
        _agg_body,
        out_type=jax.ShapeDtypeStruct((2, NPAD, D), F32),
        mesh=plsc.VectorSubcoreMesh(core_axis_name="c", subcore_axis_name="s"),
        scratch_types=[
            pltpu.VMEM((EPT,), jnp.int32),
            pltpu.VMEM((CH,), jnp.int32),
            pltpu.VMEM((CH,), jnp.int32),
            pltpu.VMEM((CH, D), F32),
            pltpu.VMEM((CH, D), F32),
            pltpu.VMEM((CH, D), F32),
            pltpu.VMEM((CH, D), F32),
            pltpu.VMEM_SHARED((NPAD, D), F32),
            pltpu.SemaphoreType.DMA,
            pltpu.SemaphoreType.DMA,
            pltpu.SemaphoreType.DMA,
            pltpu.SemaphoreType.DMA,
            pltpu.SemaphoreType.DMA,
            pltpu.SemaphoreType.DMA,
        ],
    )


def _edge_agg(h, fstk, src, dst):
    return _agg_call()(h, fstk, src, dst)


def _block_body(agg_ref, h_ref, batch_ref,
                Wc1l_ref, bc1l_ref, Wc1r_ref, W1_ref, b1_ref,
                Wc2l_ref, bc2l_ref, Wc2r_ref, W2_ref, b2_ref,
                Wcat_ref, bcat_ref, Wres_ref, bres_ref,
                hh_ref, s1_ref, c_ref):
    j = pl.program_id(0)
    h = h_ref[...]
    h1 = _dott(agg_ref[0], Wc1l_ref[...]) + bc1l_ref[...] + _dott(h, Wc1r_ref[...])
    h1 = _swish(_dott(h1, W1_ref[...]) + b1_ref[...])
    h2 = _dott(agg_ref[1], Wc2l_ref[...]) + bc2l_ref[...] + _dott(h, Wc2r_ref[...])
    h2 = _swish(_dott(h2, W2_ref[...]) + b2_ref[...])
    hh = (_dott(jnp.concatenate([h1, h2], 1), Wcat_ref[...])
          + bcat_ref[...] + h)
    for r in range(3):
        hh = _swish(_dott(hh, Wres_ref[r]) + bres_ref[r][None, :]) + hh
    hh_ref[...] = hh

    bt = batch_ref[0, 0]
    oh = _onehot(bt, NG)
    p1 = _dotT(oh, hh, hi=True)
    pc = _dotT(oh, jnp.ones_like(hh), hi=True)

    @pl.when(j == 0)
    def _():
        s1_ref[...] = p1
        c_ref[...] = pc

    @pl.when(j > 0)
    def _():
        s1_ref[...] += p1
        c_ref[...] += pc


def _block_post(agg, h, batch, Wc1l, bc1l, Wc1r, W1, b1,
                Wc2l, bc2l, Wc2r, W2, b2, Wcat, bcat, Wres, bres):
    full = lambda shape: pl.BlockSpec(shape, lambda j: tuple(0 for _ in shape))
    return pl.pallas_call(
        _block_body,
        grid=(N // TN,),
        in_specs=[pl.BlockSpec((2, TN, D), lambda j: (0, j, 0)),
                  pl.BlockSpec((TN, D), lambda j: (j, 0)),
                  pl.BlockSpec((1, 1, TN), lambda j: (j, 0, 0)),
                  full((D, D)), full((1, D)), full((D, D)),
                  full((D, D)), full((1, D)),
                  full((D, D)), full((1, D)), full((D, D)),
                  full((D, D)), full((1, D)),
                  full((D, 2 * D)), full((1, D)),
                  full((3, D, D)), full((3, D))],
        out_specs=[pl.BlockSpec((TN, D), lambda j: (j, 0)),
                   pl.BlockSpec((NG, D), lambda j: (0, 0)),
                   pl.BlockSpec((NG, D), lambda j: (0, 0))],
        out_shape=[jax.ShapeDtypeStruct((N, D), F32),
                   jax.ShapeDtypeStruct((NG, D), F32),
                   jax.ShapeDtypeStruct((NG, D), F32)],
    )(agg, h, batch, Wc1l, bc1l, Wc1r, W1, b1,
      Wc2l, bc2l, Wc2r, W2, b2, Wcat, bcat, Wres, bres)


def _varsum_body(hh_ref, s1_ref, c_ref, batch_ref, alpha_ref, s2_ref):
    j = pl.program_id(0)
    cnt = jnp.maximum(c_ref[...], 1.0)
    mean = s1_ref[...] / cnt
    bt = batch_ref[0, 0]
    oh = _onehot(bt, NG)
    sub = hh_ref[...] - alpha_ref[...] * _dot(oh, mean, hi=True)
    p2 = _dotT(oh, sub * sub, hi=True)

    @pl.when(j == 0)
    def _():
        s2_ref[...] = p2

    @pl.when(j > 0)
    def _():
        s2_ref[...] += p2


def _varsum(hh, s1, cntf, batch, alpha):
    full = lambda shape: pl.BlockSpec(shape, lambda j: tuple(0 for _ in shape))
    return pl.pallas_call(
        _varsum_body,
        grid=(N // TN,),
        in_specs=[pl.BlockSpec((TN, D), lambda j: (j, 0)),
                  full((NG, D)), full((NG, D)),
                  pl.BlockSpec((1, 1, TN), lambda j: (j, 0, 0)),
                  full((1, D))],
        out_specs=pl.BlockSpec((NG, D), lambda j: (0, 0)),
        out_shape=jax.ShapeDtypeStruct((NG, D), F32),
    )(hh, s1, cntf, batch, alpha)


def _norm_body(hh_ref, s1_ref, s2_ref, c_ref, batch_ref,
               alpha_ref, gamma_ref, beta_ref, Wfin_ref, bfin_ref,
               Wl_ref, bl_ref, x_ref, h_ref):
    cnt = jnp.maximum(c_ref[...], 1.0)
    mean = s1_ref[...] / cnt
    var = s2_ref[...] / cnt
    al = alpha_ref[...]
    bt = batch_ref[0, 0]
    oh = _onehot(bt, NG)
    mrow = _dot(oh, mean, hi=True)
    vrow = _dot(oh, var, hi=True)
    hh = hh_ref[...]
    nrm = (gamma_ref[...] * (hh - al * mrow) / jnp.sqrt(vrow + 1e-5)
           + beta_ref[...])
    x = _dott(nrm, Wfin_ref[...]) + bfin_ref[...]
    x_ref[...] = x
    h_ref[...] = _swish(_dott(x, Wl_ref[...]) + bl_ref[...])


def _norm_fin(hh, s1, s2, cntf, batch, alpha, gamma, beta, Wfin, bfin,
              Wl, bl):
    full = lambda shape: pl.BlockSpec(shape, lambda j: tuple(0 for _ in shape))
    return pl.pallas_call(
        _norm_body,
        grid=(N // TN,),
        in_specs=[pl.BlockSpec((TN, D), lambda j: (j, 0)),
                  full((NG, D)), full((NG, D)), full((NG, D)),
                  pl.BlockSpec((1, 1, TN), lambda j: (j, 0, 0)),
                  full((1, D)), full((1, D)), full((1, D)),
                  full((D, D)), full((1, D)),
                  full((D, D)), full((1, D))],
        out_specs=[pl.BlockSpec((TN, D), lambda j: (j, 0)),
                   pl.BlockSpec((TN, D), lambda j: (j, 0))],
        out_shape=[jax.ShapeDtypeStruct((N, D), F32),
                   jax.ShapeDtypeStruct((N, D), F32)],
    )(hh, s1, s2, cntf, batch, alpha, gamma, beta, Wfin, bfin, Wl, bl)


def _final_body(x_ref, batch_ref, Wout_ref, bout_ref, Wlast_ref, blast_ref,
                c_ref, energy_ref, sacc):
    j = pl.program_id(0)
    t = x_ref[...]
    for r in range(3):
        t = _swish(_dott(t, Wout_ref[r]) + bout_ref[r][None, :])
    bt = batch_ref[0, 0]
    oh = _onehot(bt, NG)
    e = _dott(t, Wlast_ref[...])
    p = _dotT(oh, e, hi=True)

    @pl.when(j == 0)
    def _():
        sacc[...] = p

    @pl.when(j > 0)
    def _():
        sacc[...] += p

    @pl.when(j == pl.num_programs(0) - 1)
    def _():
        cnt = jnp.maximum(c_ref[...], 1.0)
        energy_ref[...] = sacc[...] + _dott(cnt, blast_ref[...], hi=True)


def _final(x, batch, Wout, bout, Wlast, blast, cntf):
    full = lambda shape: pl.BlockSpec(shape, lambda j: tuple(0 for _ in shape))
    return pl.pallas_call(
        _final_body,
        grid=(N // TN,),
        in_specs=[pl.BlockSpec((TN, D), lambda j: (j, 0)),
                  pl.BlockSpec((1, 1, TN), lambda j: (j, 0, 0)),
                  full((3, D, D)), full((3, D)),
                  full((1, D)), full((1, D)), full((NG, D))],
        out_specs=pl.BlockSpec((NG, 1), lambda j: (0, 0)),
        out_shape=jax.ShapeDtypeStruct((NG, 1), F32),
        scratch_shapes=[pltpu.VMEM((NG, 1), F32)],
    )(x, batch, Wout, bout, Wlast, blast, cntf)


def kernel(emb, feature1, feature2, Wl, bl, Wf1a, Wf1b, Wf2a, Wf2b,
           Wc1l, bc1l, Wc1r, Wc2l, bc2l, Wc2r, W1, b1, W2, b2, Wcat, bcat,
           gamma, beta, alpha, Wres, bres, Wfin, bfin, Wout, bout,
           Wlast, blast, z, edge_index, batch):
    z = z.astype(jnp.int32).reshape(N // TN, 1, TN)
    src = edge_index[0].astype(jnp.int32)
    dst = edge_index[1].astype(jnp.int32)
    batch = batch.astype(jnp.int32).reshape(N // TN, 1, TN)
    r1 = lambda v: v.reshape(1, D)

    x, h = _embed(z, emb, Wl[0], r1(bl[0]))
    cntf = None
    for i in range(2):
        fstk = _fproj(feature1, feature2, Wf1a[i], Wf1b[i], Wf2a[i], Wf2b[i])
        agg = _edge_agg(h, fstk, src, dst)
        hh, s1, cntf = _block_post(
            agg, h, batch, Wc1l[i], r1(bc1l[i]), Wc1r[i], W1[i], r1(b1[i]),
            Wc2l[i], r1(bc2l[i]), Wc2r[i], W2[i], r1(b2[i]),
            Wcat[i], r1(bcat[i]), Wres[i], bres[i])
        s2 = _varsum(hh, s1, cntf, batch, r1(alpha[i]))
        nxt = (i + 1) % 2
        x, h = _norm_fin(hh, s1, s2, cntf, batch, r1(alpha[i]), r1(gamma[i]),
                         r1(beta[i]), Wfin[i], r1(bfin[i]),
                         Wl[nxt], r1(bl[nxt]))
    blastv = jnp.broadcast_to(blast.reshape(1, 1) / D, (1, D))
    return _final(x, batch, Wout, bout, Wlast, blastv, cntf)


# speedup vs baseline: 1.4757x; 1.0017x over previous
import functools
import jax
import jax.numpy as jnp
from jax import lax
from jax.experimental import pallas as pl
from jax.experimental.pallas import tpu as pltpu
from jax.experimental.pallas import tpu_sc as plsc

N = 10000
NPAD = 10240
E = 160000
D = 128
NG = 64
TN = 2000
TE = 2000
CH = 40
EPT = E // 16
NCH = EPT // CH
RPT = NPAD // 16

F32 = jnp.float32


def _swish(x):
    return x * jax.nn.sigmoid(x)


def _dott(a, b, hi=False):
    return lax.dot_general(a, b, (((1,), (1,)), ((), ())),
                           preferred_element_type=F32,
                           precision=lax.Precision.HIGHEST if hi else None)


def _dot(a, b, hi=False):
    return lax.dot_general(a, b, (((1,), (0,)), ((), ())),
                           preferred_element_type=F32,
                           precision=lax.Precision.HIGHEST if hi else None)


def _dotT(a, b, hi=False):
    return lax.dot_general(a, b, (((0,), (0,)), ((), ())),
                           preferred_element_type=F32,
                           precision=lax.Precision.HIGHEST if hi else None)


def _onehot(idx, k):
    return (idx[:, None] == lax.broadcasted_iota(jnp.int32, (1, k), 1)).astype(F32)


def _embed_body(z_ref, emb_ref, Wl_ref, bl_ref, x_ref, h_ref):
    zt = z_ref[0, 0]
    oh = _onehot(zt, 95)
    x = _swish(_dot(oh, emb_ref[...], hi=True))
    x_ref[...] = x
    h_ref[...] = _swish(_dott(x, Wl_ref[...]) + bl_ref[...])


def _embed(z, emb, Wl, bl):
    return pl.pallas_call(
        _embed_body,
        grid=(N // TN,),
        in_specs=[pl.BlockSpec((1, 1, TN), lambda j: (j, 0, 0)),
                  pl.BlockSpec((95, D), lambda j: (0, 0)),
                  pl.BlockSpec((D, D), lambda j: (0, 0)),
                  pl.BlockSpec((1, D), lambda j: (0, 0))],
        out_specs=[pl.BlockSpec((TN, D), lambda j: (j, 0)),
                   pl.BlockSpec((TN, D), lambda j: (j, 0))],
        out_shape=[jax.ShapeDtypeStruct((N, D), F32),
                   jax.ShapeDtypeStruct((N, D), F32)],
    )(z, emb, Wl, bl)


def _hproj_body(x_ref, Wl_ref, bl_ref, h_ref):
    h_ref[...] = _swish(_dott(x_ref[...], Wl_ref[...]) + bl_ref[...])


def _hproj(x, Wl, bl):
    return pl.pallas_call(
        _hproj_body,
        grid=(N // TN,),
        in_specs=[pl.BlockSpec((TN, D), lambda j: (j, 0)),
                  pl.BlockSpec((D, D), lambda j: (0, 0)),
                  pl.BlockSpec((1, D), lambda j: (0, 0))],
        out_specs=pl.BlockSpec((TN, D), lambda j: (j, 0)),
        out_shape=jax.ShapeDtypeStruct((N, D), F32),
    )(x, Wl, bl)


def _fproj_body(f1_ref, f2_ref, Wf1a_ref, Wf1b_ref, Wf2a_ref, Wf2b_ref, o_ref):
    o_ref[0] = _dott(_dott(f1_ref[...], Wf1a_ref[...]), Wf1b_ref[...])
    o_ref[1] = _dott(_dott(f2_ref[...], Wf2a_ref[...]), Wf2b_ref[...])


def _fproj(feature1, feature2, Wf1a, Wf1b, Wf2a, Wf2b):
    return pl.pallas_call(
        _fproj_body,
        grid=(E // TE,),
        in_specs=[pl.BlockSpec((TE, 12), lambda j: (j, 0)),
                  pl.BlockSpec((TE, 6), lambda j: (j, 0)),
                  pl.BlockSpec((D, 12), lambda j: (0, 0)),
                  pl.BlockSpec((D, D), lambda j: (0, 0)),
                  pl.BlockSpec((D, 6), lambda j: (0, 0)),
                  pl.BlockSpec((D, D), lambda j: (0, 0))],
        out_specs=pl.BlockSpec((2, TE, D), lambda j: (0, j, 0)),
        out_shape=jax.ShapeDtypeStruct((2, E, D), F32),
    )(feature1, feature2, Wf1a, Wf1b, Wf2a, Wf2b)


def _agg_body(h_hbm, f_hbm, src_hbm, dst_hbm, out_hbm,
              srcall, dstb0, dstb1, rows0, rows1, fbuf0, fbuf1, acc,
              sg0, sg1, sf0, sf1, sd0, sd1, ss0, ss1):
    cid = lax.axis_index("c")
    sid = lax.axis_index("s")
    tbase = sid * EPT
    dstb = (dstb0, dstb1)
    rows = (rows0, rows1)
    fbuf = (fbuf0, fbuf1)
    sg = (sg0, sg1)
    sf = (sf0, sf1)
    sd = (sd0, sd1)
    ss = (ss0, ss1)

    zero16 = jnp.zeros((16,), F32)

    def zrow(r, _):
        for c in range(8):
            fbuf0[r, pl.ds(c * 16, 16)] = zero16
        return 0
    lax.fori_loop(0, CH, zrow, 0)
    for k in range(RPT // CH):
        pltpu.sync_copy(fbuf0, acc.at[pl.ds(sid * RPT + k * CH, CH)])

    pltpu.sync_copy(src_hbm.at[pl.ds(tbase, EPT)], srcall)
    plsc.subcore_barrier()

    def prefetch(p, i):
        base = tbase + i * CH
        pltpu.async_copy(dst_hbm.at[pl.ds(base, CH)], dstb[p], sd[p])
        pltpu.async_copy(f_hbm.at[cid, pl.ds(base, CH)], fbuf[p], sf[p])
        pltpu.async_copy(h_hbm.at[srcall.at[pl.ds(i * CH, CH)]],
                         rows[p], sg[p])

    def process(p, i):
        base = tbase + i * CH
        pltpu.make_async_copy(dst_hbm.at[pl.ds(base, CH)], dstb[p],
                              sd[p]).wait()
        pltpu.make_async_copy(f_hbm.at[cid, pl.ds(base, CH)], fbuf[p],
                              sf[p]).wait()
        pltpu.make_async_copy(h_hbm.at[srcall.at[pl.ds(i * CH, CH)]],
                              rows[p], sg[p]).wait()

        def mrow(r, _):
            for c in range(8):
                s = pl.ds(c * 16, 16)
                rows[p][r, s] = rows[p][r, s] * fbuf[p][r, s]
            return 0
        lax.fori_loop(0, CH, mrow, 0)
        pltpu.async_copy(rows[p], acc.at[dstb[p]], ss[p], add=True)

    def scatter_wait(p):
        pltpu.make_async_copy(rows[p], acc.at[dstb[p]], ss[p]).wait()

    prefetch(0, 0)

    def chunk(i, _):
        for p in range(2):
            @pl.when(i % 2 == p)
            def _():
                @pl.when(i + 1 < NCH)
                def _():
                    @pl.when(i >= 1)
                    def _():
                        scatter_wait(1 - p)
                    prefetch(1 - p, i + 1)
                process(p, i)
        return 0
    lax.fori_loop(0, NCH, chunk, 0)
    scatter_wait(0)
    scatter_wait(1)

    plsc.subcore_barrier()
    pltpu.sync_copy(acc.at[pl.ds(sid * RPT, RPT)],
                    out_hbm.at[cid, pl.ds(sid * RPT, RPT)])


@functools.cache
def _agg_call():
    return pl.kernel(
        _agg_body,
        out_type=jax.ShapeDtypeStruct((2, NPAD, D), F32),
        mesh=plsc.VectorSubcoreMesh(core_axis_name="c", subcore_axis_name="s"),
        scratch_types=[
            pltpu.VMEM((EPT,), jnp.int32),
            pltpu.VMEM((CH,), jnp.int32),
            pltpu.VMEM((CH,), jnp.int32),
            pltpu.VMEM((CH, D), F32),
            pltpu.VMEM((CH, D), F32),
            pltpu.VMEM((CH, D), F32),
            pltpu.VMEM((CH, D), F32),
            pltpu.VMEM_SHARED((NPAD, D), F32),
            pltpu.SemaphoreType.DMA,
            pltpu.SemaphoreType.DMA,
            pltpu.SemaphoreType.DMA,
            pltpu.SemaphoreType.DMA,
            pltpu.SemaphoreType.DMA,
            pltpu.SemaphoreType.DMA,
            pltpu.SemaphoreType.DMA,
            pltpu.SemaphoreType.DMA,
        ],
    )


def _edge_agg(h, fstk, src, dst):
    return _agg_call()(h, fstk, src, dst)


def _block_body(agg_ref, h_ref, batch_ref,
                Wc1l_ref, bc1l_ref, Wc1r_ref, W1_ref, b1_ref,
                Wc2l_ref, bc2l_ref, Wc2r_ref, W2_ref, b2_ref,
                Wcat_ref, bcat_ref, Wres_ref, bres_ref,
                hh_ref, s1_ref, c_ref):
    j = pl.program_id(0)
    h = h_ref[...]
    h1 = _dott(agg_ref[0], Wc1l_ref[...]) + bc1l_ref[...] + _dott(h, Wc1r_ref[...])
    h1 = _swish(_dott(h1, W1_ref[...]) + b1_ref[...])
    h2 = _dott(agg_ref[1], Wc2l_ref[...]) + bc2l_ref[...] + _dott(h, Wc2r_ref[...])
    h2 = _swish(_dott(h2, W2_ref[...]) + b2_ref[...])
    hh = (_dott(jnp.concatenate([h1, h2], 1), Wcat_ref[...])
          + bcat_ref[...] + h)
    for r in range(3):
        hh = _swish(_dott(hh, Wres_ref[r]) + bres_ref[r][None, :]) + hh
    hh_ref[...] = hh

    bt = batch_ref[0, 0]
    oh = _onehot(bt, NG)
    p1 = _dotT(oh, hh, hi=True)
    pc = _dotT(oh, jnp.ones_like(hh), hi=True)

    @pl.when(j == 0)
    def _():
        s1_ref[...] = p1
        c_ref[...] = pc

    @pl.when(j > 0)
    def _():
        s1_ref[...] += p1
        c_ref[...] += pc


def _block_post(agg, h, batch, Wc1l, bc1l, Wc1r, W1, b1,
                Wc2l, bc2l, Wc2r, W2, b2, Wcat, bcat, Wres, bres):
    full = lambda shape: pl.BlockSpec(shape, lambda j: tuple(0 for _ in shape))
    return pl.pallas_call(
        _block_body,
        grid=(N // TN,),
        in_specs=[pl.BlockSpec((2, TN, D), lambda j: (0, j, 0)),
                  pl.BlockSpec((TN, D), lambda j: (j, 0)),
                  pl.BlockSpec((1, 1, TN), lambda j: (j, 0, 0)),
                  full((D, D)), full((1, D)), full((D, D)),
                  full((D, D)), full((1, D)),
                  full((D, D)), full((1, D)), full((D, D)),
                  full((D, D)), full((1, D)),
                  full((D, 2 * D)), full((1, D)),
                  full((3, D, D)), full((3, D))],
        out_specs=[pl.BlockSpec((TN, D), lambda j: (j, 0)),
                   pl.BlockSpec((NG, D), lambda j: (0, 0)),
                   pl.BlockSpec((NG, D), lambda j: (0, 0))],
        out_shape=[jax.ShapeDtypeStruct((N, D), F32),
                   jax.ShapeDtypeStruct((NG, D), F32),
                   jax.ShapeDtypeStruct((NG, D), F32)],
    )(agg, h, batch, Wc1l, bc1l, Wc1r, W1, b1,
      Wc2l, bc2l, Wc2r, W2, b2, Wcat, bcat, Wres, bres)


def _varsum_body(hh_ref, s1_ref, c_ref, batch_ref, alpha_ref, s2_ref):
    j = pl.program_id(0)
    cnt = jnp.maximum(c_ref[...], 1.0)
    mean = s1_ref[...] / cnt
    bt = batch_ref[0, 0]
    oh = _onehot(bt, NG)
    sub = hh_ref[...] - alpha_ref[...] * _dot(oh, mean, hi=True)
    p2 = _dotT(oh, sub * sub, hi=True)

    @pl.when(j == 0)
    def _():
        s2_ref[...] = p2

    @pl.when(j > 0)
    def _():
        s2_ref[...] += p2


def _varsum(hh, s1, cntf, batch, alpha):
    full = lambda shape: pl.BlockSpec(shape, lambda j: tuple(0 for _ in shape))
    return pl.pallas_call(
        _varsum_body,
        grid=(N // TN,),
        in_specs=[pl.BlockSpec((TN, D), lambda j: (j, 0)),
                  full((NG, D)), full((NG, D)),
                  pl.BlockSpec((1, 1, TN), lambda j: (j, 0, 0)),
                  full((1, D))],
        out_specs=pl.BlockSpec((NG, D), lambda j: (0, 0)),
        out_shape=jax.ShapeDtypeStruct((NG, D), F32),
    )(hh, s1, cntf, batch, alpha)


def _norm_body(hh_ref, s1_ref, s2_ref, c_ref, batch_ref,
               alpha_ref, gamma_ref, beta_ref, Wfin_ref, bfin_ref,
               Wl_ref, bl_ref, x_ref, h_ref):
    cnt = jnp.maximum(c_ref[...], 1.0)
    mean = s1_ref[...] / cnt
    var = s2_ref[...] / cnt
    al = alpha_ref[...]
    bt = batch_ref[0, 0]
    oh = _onehot(bt, NG)
    mrow = _dot(oh, mean, hi=True)
    vrow = _dot(oh, var, hi=True)
    hh = hh_ref[...]
    nrm = (gamma_ref[...] * (hh - al * mrow) / jnp.sqrt(vrow + 1e-5)
           + beta_ref[...])
    x = _dott(nrm, Wfin_ref[...]) + bfin_ref[...]
    x_ref[...] = x
    h_ref[...] = _swish(_dott(x, Wl_ref[...]) + bl_ref[...])


def _norm_fin(hh, s1, s2, cntf, batch, alpha, gamma, beta, Wfin, bfin,
              Wl, bl):
    full = lambda shape: pl.BlockSpec(shape, lambda j: tuple(0 for _ in shape))
    return pl.pallas_call(
        _norm_body,
        grid=(N // TN,),
        in_specs=[pl.BlockSpec((TN, D), lambda j: (j, 0)),
                  full((NG, D)), full((NG, D)), full((NG, D)),
                  pl.BlockSpec((1, 1, TN), lambda j: (j, 0, 0)),
                  full((1, D)), full((1, D)), full((1, D)),
                  full((D, D)), full((1, D)),
                  full((D, D)), full((1, D))],
        out_specs=[pl.BlockSpec((TN, D), lambda j: (j, 0)),
                   pl.BlockSpec((TN, D), lambda j: (j, 0))],
        out_shape=[jax.ShapeDtypeStruct((N, D), F32),
                   jax.ShapeDtypeStruct((N, D), F32)],
    )(hh, s1, s2, cntf, batch, alpha, gamma, beta, Wfin, bfin, Wl, bl)


def _final_body(x_ref, batch_ref, Wout_ref, bout_ref, Wlast_ref, blast_ref,
                c_ref, energy_ref, sacc):
    j = pl.program_id(0)
    t = x_ref[...]
    for r in range(3):
        t = _swish(_dott(t, Wout_ref[r]) + bout_ref[r][None, :])
    bt = batch_ref[0, 0]
    oh = _onehot(bt, NG)
    e = _dott(t, Wlast_ref[...])
    p = _dotT(oh, e, hi=True)

    @pl.when(j == 0)
    def _():
        sacc[...] = p

    @pl.when(j > 0)
    def _():
        sacc[...] += p

    @pl.when(j == pl.num_programs(0) - 1)
    def _():
        cnt = jnp.maximum(c_ref[...], 1.0)
        energy_ref[...] = sacc[...] + _dott(cnt, blast_ref[...], hi=True)


def _final(x, batch, Wout, bout, Wlast, blast, cntf):
    full = lambda shape: pl.BlockSpec(shape, lambda j: tuple(0 for _ in shape))
    return pl.pallas_call(
        _final_body,
        grid=(N // TN,),
        in_specs=[pl.BlockSpec((TN, D), lambda j: (j, 0)),
                  pl.BlockSpec((1, 1, TN), lambda j: (j, 0, 0)),
                  full((3, D, D)), full((3, D)),
                  full((1, D)), full((1, D)), full((NG, D))],
        out_specs=pl.BlockSpec((NG, 1), lambda j: (0, 0)),
        out_shape=jax.ShapeDtypeStruct((NG, 1), F32),
        scratch_shapes=[pltpu.VMEM((NG, 1), F32)],
    )(x, batch, Wout, bout, Wlast, blast, cntf)


def kernel(emb, feature1, feature2, Wl, bl, Wf1a, Wf1b, Wf2a, Wf2b,
           Wc1l, bc1l, Wc1r, Wc2l, bc2l, Wc2r, W1, b1, W2, b2, Wcat, bcat,
           gamma, beta, alpha, Wres, bres, Wfin, bfin, Wout, bout,
           Wlast, blast, z, edge_index, batch):
    z = z.astype(jnp.int32).reshape(N // TN, 1, TN)
    src = edge_index[0].astype(jnp.int32)
    dst = edge_index[1].astype(jnp.int32)
    batch = batch.astype(jnp.int32).reshape(N // TN, 1, TN)
    r1 = lambda v: v.reshape(1, D)

    x, h = _embed(z, emb, Wl[0], r1(bl[0]))
    cntf = None
    for i in range(2):
        fstk = _fproj(feature1, feature2, Wf1a[i], Wf1b[i], Wf2a[i], Wf2b[i])
        agg = _edge_agg(h, fstk, src, dst)
        hh, s1, cntf = _block_post(
            agg, h, batch, Wc1l[i], r1(bc1l[i]), Wc1r[i], W1[i], r1(b1[i]),
            Wc2l[i], r1(bc2l[i]), Wc2r[i], W2[i], r1(b2[i]),
            Wcat[i], r1(bcat[i]), Wres[i], bres[i])
        s2 = _varsum(hh, s1, cntf, batch, r1(alpha[i]))
        nxt = (i + 1) % 2
        x, h = _norm_fin(hh, s1, s2, cntf, batch, r1(alpha[i]), r1(gamma[i]),
                         r1(beta[i]), Wfin[i], r1(bfin[i]),
                         Wl[nxt], r1(bl[nxt]))
    blastv = jnp.broadcast_to(blast.reshape(1, 1) / D, (1, D))
    return _final(x, batch, Wout, bout, Wlast, blastv, cntf)
